# baseline (speedup 1.0000x reference)
import jax
import jax.numpy as jnp
from jax.experimental import pallas as pl
from jax.experimental.pallas import tpu as pltpu

H = 1024
E = 8
TM = 256
BN = 256
T = 4096


def _moe_body(xf_ref, w3_ref, br_ref, wg_ref, bg_ref, out_ref,
              xb_scr, wt_scr, w_scr):
    n = pl.program_id(0)
    e = pl.program_id(1)
    t = pl.program_id(2)

    @pl.when((n == 0) & (e == 0) & (t == 0))
    def _cast_x():
        xb_scr[...] = xf_ref[...].astype(jnp.bfloat16)

    @pl.when((n == 0) & (e == 0))
    def _gate():
        xs = xf_ref[pl.ds(t * TM, TM), :]
        logits = jax.lax.dot_general(
            xs, wg_ref[...], (((1,), (0,)), ((), ())),
            preferred_element_type=jnp.float32,
        ) + bg_ref[...]
        idx = jax.lax.broadcasted_iota(jnp.int32, (TM, E), 1)
        m1 = jnp.max(logits, axis=-1, keepdims=True)
        i1 = jnp.min(jnp.where(logits == m1, idx, E), axis=-1, keepdims=True)
        mask1 = idx == i1
        l2 = jnp.where(mask1, jnp.finfo(jnp.float32).min, logits)
        m2 = jnp.max(l2, axis=-1, keepdims=True)
        i2 = jnp.min(jnp.where(l2 == m2, idx, E), axis=-1, keepdims=True)
        mask2 = idx == i2
        tt = jnp.exp(m2 - m1)
        w1 = 1.0 / (1.0 + tt)
        w_scr[pl.ds(t * TM, TM), :] = (
            jnp.where(mask1, w1, 0.0) + jnp.where(mask2, 1.0 - w1, 0.0))

    @pl.when(t == 0)
    def _prep_w():
        wc = w3_ref[...].reshape(BN, H).astype(jnp.bfloat16)
        wt_scr[...] = jnp.transpose(wc, (1, 0))

    xb = xb_scr[pl.ds(t * TM, TM), :]
    y = jax.lax.dot_general(
        xb, wt_scr[...], (((1,), (0,)), ((), ())),
        preferred_element_type=jnp.float32,
    )
    wrow = w_scr[pl.ds(t * TM, TM), :]
    lane = jax.lax.broadcasted_iota(jnp.int32, (TM, E), 1)
    wcol = jnp.sum(jnp.where(lane == e, wrow, 0.0), axis=-1, keepdims=True)
    contrib = wcol * y

    @pl.when(e == 0)
    def _init():
        bias = jax.lax.dot_general(
            wrow, br_ref[...], (((1,), (0,)), ((), ())))
        out_ref[pl.ds(t * TM, TM), :] = bias + contrib

    @pl.when(e != 0)
    def _accum():
        out_ref[pl.ds(t * TM, TM), :] += contrib


def kernel(x, W_experts, b_experts, W_gate, b_gate):
    B, S, _ = x.shape
    xf = x.reshape(T, H)
    w3 = W_experts.reshape(H, E, 1, H)
    br = b_experts.reshape(H, E).T
    wg = W_gate.T
    bg = b_gate.reshape(1, E)

    out = pl.pallas_call(
        _moe_body,
        grid=(H // BN, E, T // TM),
        in_specs=[
            pl.BlockSpec((T, H), lambda n, e, t: (0, 0)),
            pl.BlockSpec((BN, 1, 1, H), lambda n, e, t: (n, e, 0, 0)),
            pl.BlockSpec((E, BN), lambda n, e, t: (0, n)),
            pl.BlockSpec((H, E), lambda n, e, t: (0, 0)),
            pl.BlockSpec((1, E), lambda n, e, t: (0, 0)),
        ],
        out_specs=pl.BlockSpec((T, BN), lambda n, e, t: (0, n)),
        out_shape=jax.ShapeDtypeStruct((T, H), jnp.float32),
        scratch_shapes=[
            pltpu.VMEM((T, H), jnp.bfloat16),
            pltpu.VMEM((H, BN), jnp.bfloat16),
            pltpu.VMEM((T, E), jnp.float32),
        ],
    )(xf, w3, br, wg, bg)
    return out.reshape(B, S, H)


# speedup vs baseline: 1.4137x; 1.4137x over previous
import jax
import jax.numpy as jnp
from jax.experimental import pallas as pl
from jax.experimental.pallas import tpu as pltpu

H = 1024
E = 8
TM = 256
T = 4096


def _moe_body(xf_ref, w3_ref, br_ref, wg_ref, bg_ref, out_ref,
              wt_scr, w_scr):
    e = pl.program_id(0)
    t = pl.program_id(1)

    xf = xf_ref[...]
    xb = xf.astype(jnp.bfloat16)

    @pl.when(e == 0)
    def _gate():
        logits = jax.lax.dot_general(
            xf, wg_ref[...], (((1,), (0,)), ((), ())),
            preferred_element_type=jnp.float32,
        ) + bg_ref[...]
        idx = jax.lax.broadcasted_iota(jnp.int32, (TM, E), 1)
        m1 = jnp.max(logits, axis=-1, keepdims=True)
        i1 = jnp.min(jnp.where(logits == m1, idx, E), axis=-1, keepdims=True)
        mask1 = idx == i1
        l2 = jnp.where(mask1, jnp.finfo(jnp.float32).min, logits)
        m2 = jnp.max(l2, axis=-1, keepdims=True)
        i2 = jnp.min(jnp.where(l2 == m2, idx, E), axis=-1, keepdims=True)
        mask2 = idx == i2
        tt = jnp.exp(m2 - m1)
        w1 = 1.0 / (1.0 + tt)
        w_scr[pl.ds(t * TM, TM), :] = (
            jnp.where(mask1, w1, 0.0) + jnp.where(mask2, 1.0 - w1, 0.0))

    @pl.when(t == 0)
    def _prep_w():
        wc = w3_ref[...].reshape(H, H).astype(jnp.bfloat16)
        wt_scr[...] = jnp.transpose(wc, (1, 0))

    y = jax.lax.dot_general(
        xb, wt_scr[...], (((1,), (0,)), ((), ())),
        preferred_element_type=jnp.float32,
    )
    wrow = w_scr[pl.ds(t * TM, TM), :]
    lane = jax.lax.broadcasted_iota(jnp.int32, (TM, E), 1)
    wcol = jnp.sum(jnp.where(lane == e, wrow, 0.0), axis=-1, keepdims=True)
    contrib = wcol * y

    @pl.when(e == 0)
    def _init():
        bias = jax.lax.dot_general(
            wrow, br_ref[...], (((1,), (0,)), ((), ())))
        out_ref[pl.ds(t * TM, TM), :] = bias + contrib

    @pl.when(e != 0)
    def _accum():
        out_ref[pl.ds(t * TM, TM), :] += contrib


def kernel(x, W_experts, b_experts, W_gate, b_gate):
    B, S, _ = x.shape
    xf = x.reshape(T, H)
    w3 = W_experts.reshape(H, E, 1, H)
    br = b_experts.reshape(H, E).T
    wg = W_gate.T
    bg = b_gate.reshape(1, E)

    out = pl.pallas_call(
        _moe_body,
        grid=(E, T // TM),
        in_specs=[
            pl.BlockSpec((TM, H), lambda e, t: (t, 0)),
            pl.BlockSpec((H, 1, 1, H), lambda e, t: (0, e, 0, 0)),
            pl.BlockSpec((E, H), lambda e, t: (0, 0)),
            pl.BlockSpec((H, E), lambda e, t: (0, 0)),
            pl.BlockSpec((1, E), lambda e, t: (0, 0)),
        ],
        out_specs=pl.BlockSpec((T, H), lambda e, t: (0, 0)),
        out_shape=jax.ShapeDtypeStruct((T, H), jnp.float32),
        scratch_shapes=[
            pltpu.VMEM((H, H), jnp.bfloat16),
            pltpu.VMEM((T, E), jnp.float32),
        ],
    )(xf, w3, br, wg, bg)
    return out.reshape(B, S, H)
